# baseline (speedup 1.0000x reference)
import functools

import jax
import jax.numpy as jnp
from jax import lax
from jax.experimental import pallas as pl
from jax.experimental.pallas import tpu as pltpu
from jax.experimental.pallas import tpu_sc as plsc

_CHUNK = 128
_NW = 32


@functools.partial(jax.jit, static_argnames=("J", "Bt", "D"))
def _embed(xt3, table, *, J, Bt, D):
    bw = Bt // _NW
    hb = bw // _CHUNK
    nchunk = J * hb

    mesh = plsc.VectorSubcoreMesh(core_axis_name="c", subcore_axis_name="s")

    @functools.partial(
        pl.kernel,
        mesh=mesh,
        out_type=jax.ShapeDtypeStruct((J, D, Bt), jnp.float32),
        compiler_params=pltpu.CompilerParams(use_tc_tiling_on_sc=False,
                                             needs_layout_passes=False),
        scratch_types=(
            [pltpu.VMEM((J, hb, _CHUNK), jnp.int32),
             pltpu.VMEM((2, _CHUNK, D), jnp.float32),
             pltpu.VMEM((2, D, _CHUNK), jnp.float32)]
            + [pltpu.SemaphoreType.DMA] * 4
        ),
    )
    def emb(table_hbm, x_hbm, out_hbm, idx_v, gbuf, tbuf, *sems):
        gsem = sems[:2]
        osem = sems[2:]
        wid = lax.axis_index("s") * 2 + lax.axis_index("c")
        b0 = wid * bw
        pltpu.sync_copy(x_hbm.at[:, pl.ds(wid * hb, hb), :], idx_v)

        def jh(i):
            j = i // hb
            return j, i - j * hb

        def fire_gather(i, p):
            j, h = jh(i)
            pltpu.async_copy(table_hbm.at[idx_v.at[j, h]], gbuf.at[p],
                             gsem[p])

        def wait_gather(i, p):
            j, h = jh(i)
            pltpu.make_async_copy(table_hbm.at[idx_v.at[j, h]], gbuf.at[p],
                                  gsem[p]).wait()

        def out_slab(i):
            j, h = jh(i)
            return out_hbm.at[j, :, pl.ds(b0 + h * _CHUNK, _CHUNK)]

        def fire_out(i, p):
            pltpu.async_copy(tbuf.at[p], out_slab(i), osem[p])

        def wait_out(i, p):
            pltpu.make_async_copy(tbuf.at[p], out_slab(i), osem[p]).wait()

        row_iota = lax.iota(jnp.int32, 16)

        def transpose(p):
            src = gbuf.at[p]
            dst = tbuf.at[p]
            for c in range(D):
                cc = jnp.full((16,), c, jnp.int32)
                for kk in range(_CHUNK // 16):
                    v = plsc.load_gather(src, [row_iota + (kk * 16), cc])
                    dst[c, pl.ds(kk * 16, 16)] = v

        def step(i, p, first, last):
            wait_gather(i, p)
            if not first:
                wait_out(i - 2, p)
            transpose(p)
            if not last:
                fire_gather(i + 2, p)
            fire_out(i, p)

        fire_gather(0, 0)
        fire_gather(1, 1)
        step(0, 0, True, False)
        step(1, 1, True, False)

        def group(g, carry):
            i = g * 2
            step(i, 0, False, False)
            step(i + 1, 1, False, False)
            return carry

        lax.fori_loop(1, nchunk // 2 - 1, group, 0)

        step(nchunk - 2, 0, False, True)
        step(nchunk - 1, 1, False, True)
        wait_out(nchunk - 2, 0)
        wait_out(nchunk - 1, 1)

    return emb(table, xt3)


def kernel(x, table):
    B, J = x.shape
    D = table.shape[1]
    xt = jnp.transpose(x).astype(jnp.int32)
    xt3 = xt.reshape(J, B // _CHUNK, _CHUNK)
    out_phys = _embed(xt3, table, J=J, Bt=B, D=D)
    return jnp.transpose(out_phys, (2, 0, 1))


# speedup vs baseline: 1.5233x; 1.5233x over previous
import functools

import jax
import jax.numpy as jnp
from jax import lax
from jax.experimental import pallas as pl
from jax.experimental.pallas import tpu as pltpu
from jax.experimental.pallas import tpu_sc as plsc

_CHUNK = 128
_NW = 32


@functools.partial(jax.jit, static_argnames=("J", "Bt", "D"))
def _embed(xt3, table, *, J, Bt, D):
    bw = Bt // _NW
    hb = bw // _CHUNK
    nchunk = J * hb

    mesh = plsc.VectorSubcoreMesh(core_axis_name="c", subcore_axis_name="s")

    @functools.partial(
        pl.kernel,
        mesh=mesh,
        out_type=jax.ShapeDtypeStruct((J, D, Bt), jnp.float32),
        compiler_params=pltpu.CompilerParams(use_tc_tiling_on_sc=False,
                                             needs_layout_passes=False),
        scratch_types=(
            [pltpu.VMEM((J, hb, _CHUNK), jnp.int32),
             pltpu.VMEM((2, _CHUNK, D), jnp.float32),
             pltpu.VMEM((2, D, _CHUNK), jnp.float32)]
            + [pltpu.SemaphoreType.DMA] * 4
        ),
    )
    def emb(table_hbm, x_hbm, out_hbm, idx_v, gbuf, tbuf, *sems):
        gsem = sems[:2]
        osem = sems[2:]
        wid = lax.axis_index("s") * 2 + lax.axis_index("c")
        b0 = wid * bw
        pltpu.sync_copy(x_hbm.at[:, pl.ds(wid * hb, hb), :], idx_v)

        def jh(i):
            j = i // hb
            return j, i - j * hb

        def fire_gather(i, p):
            j, h = jh(i)
            pltpu.async_copy(table_hbm.at[idx_v.at[j, h]], gbuf.at[p],
                             gsem[p])

        def wait_gather(i, p):
            j, h = jh(i)
            pltpu.make_async_copy(table_hbm.at[idx_v.at[j, h]], gbuf.at[p],
                                  gsem[p]).wait()

        def out_slab(i):
            j, h = jh(i)
            return out_hbm.at[j, :, pl.ds(b0 + h * _CHUNK, _CHUNK)]

        def fire_out(i, p):
            pltpu.async_copy(tbuf.at[p], out_slab(i), osem[p])

        def wait_out(i, p):
            pltpu.make_async_copy(tbuf.at[p], out_slab(i), osem[p]).wait()

        row_iota = lax.iota(jnp.int32, 16)

        def transpose(p):
            src = gbuf.at[p]
            dst = tbuf.at[p]

            @plsc.parallel_loop(0, D, unroll=4)
            def _(c):
                cc = jnp.zeros((16,), jnp.int32) + c
                for kk in range(_CHUNK // 16):
                    v = plsc.load_gather(src, [row_iota + (kk * 16), cc])
                    dst[c, pl.ds(kk * 16, 16)] = v

        def step(i, p, first, last):
            wait_gather(i, p)
            if not first:
                wait_out(i - 2, p)
            transpose(p)
            if not last:
                fire_gather(i + 2, p)
            fire_out(i, p)

        fire_gather(0, 0)
        fire_gather(1, 1)
        step(0, 0, True, False)
        step(1, 1, True, False)

        def group(g, carry):
            i = g * 2
            step(i, 0, False, False)
            step(i + 1, 1, False, False)
            return carry

        lax.fori_loop(1, nchunk // 2 - 1, group, 0)

        step(nchunk - 2, 0, False, True)
        step(nchunk - 1, 1, False, True)
        wait_out(nchunk - 2, 0)
        wait_out(nchunk - 1, 1)

    return emb(table, xt3)


def kernel(x, table):
    B, J = x.shape
    D = table.shape[1]
    xt = jnp.transpose(x).astype(jnp.int32)
    xt3 = xt.reshape(J, B // _CHUNK, _CHUNK)
    out_phys = _embed(xt3, table, J=J, Bt=B, D=D)
    return jnp.transpose(out_phys, (2, 0, 1))
